# Initial kernel scaffold; baseline (speedup 1.0000x reference)
import jax
import jax.numpy as jnp
from jax.experimental import pallas as pl


def kernel(batch_confidence_maps, record_len, pairwise_t_matrix, gauss_w, gauss_b):
    raise NotImplementedError("write your pallas kernel here")


# speedup vs baseline: 12.3540x; 12.3540x over previous
import functools

import jax
import jax.numpy as jnp
from jax.experimental import pallas as pl
from jax.experimental.pallas import tpu as pltpu

THRE = 0.03
NUM_BLOCKS_THRES = 0.5
KSIZE = 5
PAD = (KSIZE - 1) // 2


def _comm_body(w_ref, b_ref, m_ref, comm_ref, mask_ref, sum_ref, pad_ref, *, h, w, kth):
    ori = jax.nn.sigmoid(jnp.maximum(m_ref[0, 0, :, :], m_ref[0, 1, :, :]))

    pad_ref[...] = jnp.zeros_like(pad_ref)
    pad_ref[PAD:PAD + h, PAD:PAD + w] = ori.astype(jnp.bfloat16).astype(jnp.float32)

    zero = jnp.zeros((h, w), dtype=jnp.float32)
    s, c = zero, zero
    groups = []
    for k in range(KSIZE * KSIZE):
        dy, dx = divmod(k, KSIZE)
        p = w_ref[dy, dx] * pad_ref[dy:dy + h, dx:dx + w]
        t = s + p
        p2 = t - s
        c = c + ((s - (t - p2)) + (p - p2))
        s = t
        if k % 8 == 7 or k == KSIZE * KSIZE - 1:
            groups.append(s + c)
            s, c = zero, zero
    sm = groups[0]
    for g in groups[1:]:
        sm = sm + g
    sm = sm + b_ref[0]

    cnt = jnp.sum((sm > THRE).astype(jnp.int32))

    bits = jax.lax.bitcast_convert_type(sm, jnp.int32)

    def cond(carry):
        lo, hi = carry
        return lo < hi

    def body(carry):
        lo, hi = carry
        mid = lo + (hi - lo + 1) // 2
        c = jnp.sum((bits >= mid).astype(jnp.int32))
        return jnp.where(c >= kth, mid, lo), jnp.where(c >= kth, hi, mid - 1)

    lo0 = jnp.int32(0)
    hi0 = jnp.max(bits)
    tau, _ = jax.lax.while_loop(cond, body, (lo0, hi0))

    topk_mask = (bits >= tau).astype(jnp.float32)
    thre_mask = (sm > THRE).astype(jnp.float32)
    use_topk = cnt > kth
    mask = jnp.where(use_topk, topk_mask, thre_mask)

    mask_ref[0, :, :] = mask
    comm_ref[0, :, :] = ori * mask
    sum_ref[...] = jnp.full((1, 1, 1), jnp.sum(mask), dtype=jnp.float32)


def kernel(batch_confidence_maps, record_len, pairwise_t_matrix, gauss_w, gauss_b):
    B, Lk, C, H, W = batch_confidence_maps.shape
    n_maps = B * Lk
    kth = int(H * W * NUM_BLOCKS_THRES)

    maps = batch_confidence_maps.reshape(n_maps, C, H, W)
    wu = jax.lax.bitcast_convert_type(gauss_w.reshape(KSIZE, KSIZE), jnp.uint32)
    wr = ((wu >> 16) & jnp.uint32(1)) + jnp.uint32(0x7FFF)
    w2d = jax.lax.bitcast_convert_type((wu + wr) & jnp.uint32(0xFFFF0000), jnp.float32)
    b1 = gauss_b.reshape(1)

    body = functools.partial(_comm_body, h=H, w=W, kth=kth)

    comm, masks, sums = pl.pallas_call(
        body,
        grid=(n_maps,),
        in_specs=[
            pl.BlockSpec(memory_space=pltpu.SMEM),
            pl.BlockSpec(memory_space=pltpu.SMEM),
            pl.BlockSpec((1, C, H, W), lambda i: (i, 0, 0, 0)),
        ],
        out_specs=[
            pl.BlockSpec((1, H, W), lambda i: (i, 0, 0)),
            pl.BlockSpec((1, H, W), lambda i: (i, 0, 0)),
            pl.BlockSpec((1, 1, 1), lambda i: (i, 0, 0)),
        ],
        out_shape=[
            jax.ShapeDtypeStruct((n_maps, H, W), jnp.float32),
            jax.ShapeDtypeStruct((n_maps, H, W), jnp.float32),
            jax.ShapeDtypeStruct((n_maps, 1, 1), jnp.float32),
        ],
        scratch_shapes=[pltpu.VMEM((H + 2 * PAD, W + 2 * PAD), jnp.float32)],
        compiler_params=pltpu.CompilerParams(
            dimension_semantics=("parallel",),
        ),
    )(w2d, b1, maps)

    comm_maps = comm.reshape(B, Lk, 1, H, W)
    masks_out = masks.reshape(B, Lk, 1, H, W)
    rate = jnp.sum(sums.reshape(B, Lk)[:, 0]) / B
    return comm_maps, masks_out, rate


# speedup vs baseline: 13.1776x; 1.0667x over previous
import functools

import jax
import jax.numpy as jnp
from jax.experimental import pallas as pl
from jax.experimental.pallas import tpu as pltpu

THRE = 0.03
NUM_BLOCKS_THRES = 0.5
KSIZE = 5
PAD = (KSIZE - 1) // 2


def _comm_body(w_ref, b_ref, m_ref, comm_ref, mask_ref, sum_ref, pad_ref, *, h, w, kth):
    ori = jax.nn.sigmoid(jnp.maximum(m_ref[0, 0, :, :], m_ref[0, 1, :, :]))

    pad_ref[...] = jnp.zeros_like(pad_ref)
    pad_ref[PAD:PAD + h, PAD:PAD + w] = ori.astype(jnp.bfloat16).astype(jnp.float32)

    zero = jnp.zeros((h, w), dtype=jnp.float32)
    s, c = zero, zero
    groups = []
    for k in range(KSIZE * KSIZE):
        dy, dx = divmod(k, KSIZE)
        p = w_ref[dy, dx] * pad_ref[dy:dy + h, dx:dx + w]
        t = s + p
        p2 = t - s
        c = c + ((s - (t - p2)) + (p - p2))
        s = t
        if k % 8 == 7 or k == KSIZE * KSIZE - 1:
            groups.append(s + c)
            s, c = zero, zero
    sm = groups[0]
    for g in groups[1:]:
        sm = sm + g
    sm = sm + b_ref[0]

    cnt = jnp.sum((sm > THRE).astype(jnp.int32))

    bits = jax.lax.bitcast_convert_type(sm, jnp.int32)

    def cond(carry):
        lo, hi = carry
        return lo < hi

    def body(carry):
        lo, hi = carry
        mid = lo + (hi - lo + 1) // 2
        c = jnp.sum((bits >= mid).astype(jnp.int32))
        return jnp.where(c >= kth, mid, lo), jnp.where(c >= kth, hi, mid - 1)

    lo0 = jax.lax.bitcast_convert_type(jnp.float32(THRE), jnp.int32)
    hi0 = jnp.max(bits)
    tau, _ = jax.lax.while_loop(cond, body, (lo0, hi0))

    topk_mask = (bits >= tau).astype(jnp.float32)
    thre_mask = (sm > THRE).astype(jnp.float32)
    use_topk = cnt > kth
    mask = jnp.where(use_topk, topk_mask, thre_mask)

    mask_ref[0, :, :] = mask
    comm_ref[0, :, :] = ori * mask
    sum_ref[...] = jnp.full((1, 1, 1), jnp.sum(mask), dtype=jnp.float32)


def kernel(batch_confidence_maps, record_len, pairwise_t_matrix, gauss_w, gauss_b):
    B, Lk, C, H, W = batch_confidence_maps.shape
    n_maps = B * Lk
    kth = int(H * W * NUM_BLOCKS_THRES)

    maps = batch_confidence_maps.reshape(n_maps, C, H, W)
    wu = jax.lax.bitcast_convert_type(gauss_w.reshape(KSIZE, KSIZE), jnp.uint32)
    wr = ((wu >> 16) & jnp.uint32(1)) + jnp.uint32(0x7FFF)
    w2d = jax.lax.bitcast_convert_type((wu + wr) & jnp.uint32(0xFFFF0000), jnp.float32)
    b1 = gauss_b.reshape(1)

    body = functools.partial(_comm_body, h=H, w=W, kth=kth)

    comm, masks, sums = pl.pallas_call(
        body,
        grid=(n_maps,),
        in_specs=[
            pl.BlockSpec(memory_space=pltpu.SMEM),
            pl.BlockSpec(memory_space=pltpu.SMEM),
            pl.BlockSpec((1, C, H, W), lambda i: (i, 0, 0, 0)),
        ],
        out_specs=[
            pl.BlockSpec((1, H, W), lambda i: (i, 0, 0)),
            pl.BlockSpec((1, H, W), lambda i: (i, 0, 0)),
            pl.BlockSpec((1, 1, 1), lambda i: (i, 0, 0)),
        ],
        out_shape=[
            jax.ShapeDtypeStruct((n_maps, H, W), jnp.float32),
            jax.ShapeDtypeStruct((n_maps, H, W), jnp.float32),
            jax.ShapeDtypeStruct((n_maps, 1, 1), jnp.float32),
        ],
        scratch_shapes=[pltpu.VMEM((H + 2 * PAD, W + 2 * PAD), jnp.float32)],
        compiler_params=pltpu.CompilerParams(
            dimension_semantics=("parallel",),
        ),
    )(w2d, b1, maps)

    comm_maps = comm.reshape(B, Lk, 1, H, W)
    masks_out = masks.reshape(B, Lk, 1, H, W)
    rate = jnp.sum(sums.reshape(B, Lk)[:, 0]) / B
    return comm_maps, masks_out, rate
